# Initial kernel scaffold; baseline (speedup 1.0000x reference)
import jax
import jax.numpy as jnp
from jax.experimental import pallas as pl


def kernel(x, edge_index, W1, b1, W2, b2):
    raise NotImplementedError("write your pallas kernel here")


# speedup vs baseline: 42.5299x; 42.5299x over previous
import functools

import jax
import jax.numpy as jnp
from jax import lax
from jax.experimental import pallas as pl
from jax.experimental.pallas import tpu as pltpu
from jax.experimental.pallas import tpu_sc as plsc

N_NODES = 100000
N_EDGES = 3200000
FEAT = 20
HID = 32
OUT = 2

NC = 2
NS = 16
NW = NC * NS
NPAD = 100352
ROWS_PER_SUB = NPAD // NS
GRID_R = NPAD // 128
FH = 16

CHUNK = 1024
N_ITERS = 98
E_PER_W = CHUNK * N_ITERS
EPAD = E_PER_W * NW
E_PER_SUB = EPAD // NS
L1_ITERS = E_PER_SUB // CHUNK

_mesh = plsc.VectorSubcoreMesh(core_axis_name="c", subcore_axis_name="s")
_cparams = pltpu.CompilerParams(use_tc_tiling_on_sc=False)


@functools.partial(
    pl.kernel,
    out_type=jax.ShapeDtypeStruct((NC, NPAD), jnp.float32),
    mesh=_mesh,
    compiler_params=_cparams,
    scratch_types=[
        pltpu.VMEM((CHUNK,), jnp.int32),
        pltpu.VMEM((CHUNK,), jnp.float32),
        pltpu.VMEM_SHARED((NPAD,), jnp.float32),
    ],
)
def _sc_degree(dst_hbm, ones_hbm, zeros_hbm, out_hbm, dst_v, ones_v, acc_sh):
    cid = lax.axis_index("c")
    sid = lax.axis_index("s")
    wid = sid * NC + cid
    row0 = sid * ROWS_PER_SUB
    pltpu.sync_copy(zeros_hbm.at[pl.ds(row0, ROWS_PER_SUB)],
                    acc_sh.at[pl.ds(row0, ROWS_PER_SUB)])
    pltpu.sync_copy(ones_hbm, ones_v)
    plsc.subcore_barrier()

    @pl.loop(0, N_ITERS)
    def _(it):
        base = wid * E_PER_W + it * CHUNK
        pltpu.sync_copy(dst_hbm.at[pl.ds(base, CHUNK)], dst_v)
        pltpu.sync_copy(ones_v, acc_sh.at[dst_v], add=True)

    plsc.subcore_barrier()
    pltpu.sync_copy(acc_sh.at[pl.ds(row0, ROWS_PER_SUB)],
                    out_hbm.at[cid].at[pl.ds(row0, ROWS_PER_SUB)])


@functools.partial(
    pl.kernel,
    out_type=jax.ShapeDtypeStruct((NC, NPAD, FH), jnp.float32),
    mesh=_mesh,
    compiler_params=_cparams,
    scratch_types=[
        pltpu.VMEM((CHUNK,), jnp.int32),
        pltpu.VMEM((CHUNK,), jnp.int32),
        pltpu.VMEM((CHUNK, FH), jnp.float32),
        pltpu.VMEM_SHARED((NPAD, FH), jnp.float32),
        pltpu.SemaphoreType.DMA,
    ],
)
def _sc_agg_l1(src_hbm, dst_hbm, ya_hbm, yb_hbm, zeros_hbm, out_hbm,
               src_v, dst_v, rows_v, acc_sh, sem):
    cid = lax.axis_index("c")
    sid = lax.axis_index("s")
    row0 = sid * ROWS_PER_SUB
    pltpu.sync_copy(zeros_hbm.at[pl.ds(row0, ROWS_PER_SUB)],
                    acc_sh.at[pl.ds(row0, ROWS_PER_SUB)])
    plsc.subcore_barrier()

    def edge_loop(y_hbm):
        @pl.loop(0, L1_ITERS)
        def _(it):
            base = sid * E_PER_SUB + it * CHUNK
            pltpu.sync_copy(src_hbm.at[pl.ds(base, CHUNK)], src_v)
            pltpu.sync_copy(dst_hbm.at[pl.ds(base, CHUNK)], dst_v)
            pltpu.async_copy(y_hbm.at[src_v], rows_v, sem).wait()
            pltpu.sync_copy(rows_v, acc_sh.at[dst_v], add=True)

    @pl.when(cid == 0)
    def _():
        edge_loop(ya_hbm)

    @pl.when(cid == 1)
    def _():
        edge_loop(yb_hbm)

    plsc.subcore_barrier()
    pltpu.sync_copy(acc_sh.at[pl.ds(row0, ROWS_PER_SUB)],
                    out_hbm.at[cid].at[pl.ds(row0, ROWS_PER_SUB)])


@functools.partial(
    pl.kernel,
    out_type=jax.ShapeDtypeStruct((NC, NPAD, FH), jnp.float32),
    mesh=_mesh,
    compiler_params=_cparams,
    scratch_types=[
        pltpu.VMEM((CHUNK,), jnp.int32),
        pltpu.VMEM((CHUNK,), jnp.int32),
        pltpu.VMEM((CHUNK, FH), jnp.float32),
        pltpu.VMEM_SHARED((NPAD, FH), jnp.float32),
        pltpu.SemaphoreType.DMA,
    ],
)
def _sc_agg_l2(src_hbm, dst_hbm, y_hbm, zeros_hbm, out_hbm,
               src_v, dst_v, rows_v, acc_sh, sem):
    cid = lax.axis_index("c")
    sid = lax.axis_index("s")
    wid = sid * NC + cid
    row0 = sid * ROWS_PER_SUB
    pltpu.sync_copy(zeros_hbm.at[pl.ds(row0, ROWS_PER_SUB)],
                    acc_sh.at[pl.ds(row0, ROWS_PER_SUB)])
    plsc.subcore_barrier()

    @pl.loop(0, N_ITERS)
    def _(it):
        base = wid * E_PER_W + it * CHUNK
        pltpu.sync_copy(src_hbm.at[pl.ds(base, CHUNK)], src_v)
        pltpu.sync_copy(dst_hbm.at[pl.ds(base, CHUNK)], dst_v)
        pltpu.async_copy(y_hbm.at[src_v], rows_v, sem).wait()
        pltpu.sync_copy(rows_v, acc_sh.at[dst_v], add=True)

    plsc.subcore_barrier()
    pltpu.sync_copy(acc_sh.at[pl.ds(row0, ROWS_PER_SUB)],
                    out_hbm.at[cid].at[pl.ds(row0, ROWS_PER_SUB)])


BLK = 2048
TGRID = NPAD // BLK


def _row_spec(w):
    return pl.BlockSpec((BLK, w), lambda i: (i, 0))


def _rep_spec(shape):
    return pl.BlockSpec(shape, lambda i: (0, 0))


def _tc_rsqrt_body(deg_ref, o_ref):
    o_ref[...] = lax.rsqrt(deg_ref[0] + deg_ref[1] + 1.0)


_tc_rsqrt = pl.pallas_call(
    _tc_rsqrt_body,
    out_shape=jax.ShapeDtypeStruct((GRID_R, 128), jnp.float32),
)


def _tc_scale_body(x_ref, rep_ref, o_ref):
    o_ref[...] = x_ref[...] * rep_ref[...]


_tc_scale = pl.pallas_call(
    _tc_scale_body,
    grid=(TGRID,),
    in_specs=[_row_spec(2 * FH), _row_spec(2 * FH)],
    out_specs=_row_spec(2 * FH),
    out_shape=jax.ShapeDtypeStruct((NPAD, 2 * FH), jnp.float32),
)


def _tc_dense_body(a0_ref, a1_ref, xs_ref, rep32_ref, w1_ref, b1_ref,
                   w2_ref, rep16_ref, o_ref):
    agg = (jnp.concatenate([a0_ref[...], a1_ref[...]], axis=1)
           + xs_ref[...]) * rep32_ref[...]
    h1 = jnp.maximum(
        jnp.dot(agg, w1_ref[...], preferred_element_type=jnp.float32)
        + b1_ref[...], 0.0)
    h2 = jnp.dot(h1, w2_ref[...], preferred_element_type=jnp.float32)
    o_ref[...] = h2 * rep16_ref[...]


_tc_dense = pl.pallas_call(
    _tc_dense_body,
    grid=(TGRID,),
    in_specs=[_row_spec(FH), _row_spec(FH), _row_spec(2 * FH),
              _row_spec(2 * FH), _rep_spec((2 * FH, HID)),
              _rep_spec((1, HID)), _rep_spec((HID, FH)), _row_spec(FH)],
    out_specs=_row_spec(FH),
    out_shape=jax.ShapeDtypeStruct((NPAD, FH), jnp.float32),
)


def _tc_final_body(a0_ref, a1_ref, h2s_ref, rep16_ref, b2_ref, o_ref):
    o_ref[...] = ((a0_ref[...] + a1_ref[...] + h2s_ref[...])
                  * rep16_ref[...] + b2_ref[...])


_tc_final = pl.pallas_call(
    _tc_final_body,
    grid=(TGRID,),
    in_specs=[_row_spec(FH), _row_spec(FH), _row_spec(FH), _row_spec(FH),
              _rep_spec((1, FH))],
    out_specs=_row_spec(FH),
    out_shape=jax.ShapeDtypeStruct((NPAD, FH), jnp.float32),
)


def kernel(x, edge_index, W1, b1, W2, b2):
    n_pad_e = EPAD - N_EDGES
    pad_rows = N_NODES + (jnp.arange(n_pad_e, dtype=jnp.int32)
                          % (NPAD - N_NODES))
    src = jnp.concatenate([edge_index[0].astype(jnp.int32), pad_rows])
    dst = jnp.concatenate([edge_index[1].astype(jnp.int32), pad_rows])

    ones_chunk = jnp.ones((CHUNK,), jnp.float32)
    z1 = jnp.zeros((NPAD,), jnp.float32)
    z16 = jnp.zeros((NPAD, FH), jnp.float32)

    deg_pair = _sc_degree(dst, ones_chunk, z1)
    dinv_grid = _tc_rsqrt(deg_pair.reshape(NC, GRID_R, 128))
    dinv = dinv_grid.reshape(NPAD)
    rep32 = jnp.broadcast_to(dinv[:, None], (NPAD, 2 * FH))
    rep16 = jnp.broadcast_to(dinv[:, None], (NPAD, FH))

    xpad = jnp.pad(x, ((0, NPAD - N_NODES), (0, 2 * FH - FEAT)))
    xs = _tc_scale(xpad, rep32)

    acc1 = _sc_agg_l1(src, dst, xs[:, :FH], xs[:, FH:], z16)

    w1p = jnp.pad(W1, ((0, 2 * FH - FEAT), (0, 0)))
    w2p = jnp.pad(W2, ((0, 0), (0, FH - OUT)))
    h2s = _tc_dense(acc1[0], acc1[1], xs, rep32,
                    w1p, b1.reshape(1, HID), w2p, rep16)

    acc2 = _sc_agg_l2(src, dst, h2s, z16)

    b2p = jnp.pad(b2, (0, FH - OUT)).reshape(1, FH)
    out = _tc_final(acc2[0], acc2[1], h2s, rep16, b2p)
    return out[:N_NODES, :OUT]


# speedup vs baseline: 47.0505x; 1.1063x over previous
import functools

import jax
import jax.numpy as jnp
from jax import lax
from jax.experimental import pallas as pl
from jax.experimental.pallas import tpu as pltpu
from jax.experimental.pallas import tpu_sc as plsc

N_NODES = 100000
N_EDGES = 3200000
FEAT = 20
HID = 32
OUT = 2

NC = 2
NS = 16
NW = NC * NS
NPAD = 100352
ROWS_PER_SUB = NPAD // NS
FH = 16

CHUNK = 1024
E_PER_W = 100352
W_ITERS = E_PER_W // CHUNK
W_ITERS_LAST = (N_EDGES - (NW - 1) * E_PER_W) // CHUNK
E_PER_SUB = 200704
S_ITERS = E_PER_SUB // CHUNK
S_ITERS_LAST = (N_EDGES - (NS - 1) * E_PER_SUB) // CHUNK

_mesh = plsc.VectorSubcoreMesh(core_axis_name="c", subcore_axis_name="s")
_cparams = pltpu.CompilerParams(use_tc_tiling_on_sc=False,
                                needs_layout_passes=False)


@functools.partial(
    pl.kernel,
    out_type=jax.ShapeDtypeStruct((NC, NPAD, FH), jnp.float32),
    mesh=_mesh,
    compiler_params=_cparams,
    scratch_types=[
        pltpu.VMEM((CHUNK,), jnp.int32),
        pltpu.VMEM((CHUNK,), jnp.float32),
        pltpu.VMEM((ROWS_PER_SUB,), jnp.float32),
        pltpu.VMEM((ROWS_PER_SUB, FH), jnp.float32),
        pltpu.VMEM_SHARED((NPAD,), jnp.float32),
    ],
)
def _sc_degree(dst_hbm, ones_hbm, zeros_hbm, out_hbm,
               dst_v, ones_v, slice_v, rep_v, acc_sh):
    cid = lax.axis_index("c")
    sid = lax.axis_index("s")
    wid = sid * NC + cid
    row0 = sid * ROWS_PER_SUB
    pltpu.sync_copy(zeros_hbm.at[pl.ds(row0, ROWS_PER_SUB)],
                    acc_sh.at[pl.ds(row0, ROWS_PER_SUB)])
    pltpu.sync_copy(ones_hbm, ones_v)
    plsc.subcore_barrier()

    nit = jnp.where(wid == NW - 1, W_ITERS_LAST, W_ITERS)

    @pl.loop(0, nit)
    def _(it):
        base = wid * E_PER_W + it * CHUNK
        pltpu.sync_copy(dst_hbm.at[pl.ds(base, CHUNK)], dst_v)
        pltpu.sync_copy(ones_v, acc_sh.at[dst_v], add=True)

    plsc.subcore_barrier()
    pltpu.sync_copy(acc_sh.at[pl.ds(row0, ROWS_PER_SUB)], slice_v)
    iota16 = lax.iota(jnp.int32, 16)

    @pl.loop(0, ROWS_PER_SUB, step=16)
    def _(r0):
        vals = slice_v[pl.ds(r0, 16)]
        rows = iota16 + r0
        for j in range(FH):
            plsc.store_scatter(rep_v, [rows, jnp.full((16,), j, jnp.int32)],
                               vals)

    pltpu.sync_copy(rep_v, out_hbm.at[cid].at[pl.ds(row0, ROWS_PER_SUB)])


@functools.partial(
    pl.kernel,
    out_type=jax.ShapeDtypeStruct((NC, NPAD, FH), jnp.float32),
    mesh=_mesh,
    compiler_params=_cparams,
    scratch_types=[
        pltpu.VMEM((CHUNK,), jnp.int32),
        pltpu.VMEM((CHUNK,), jnp.int32),
        pltpu.VMEM((CHUNK, FH), jnp.float32),
        pltpu.VMEM_SHARED((NPAD, FH), jnp.float32),
        pltpu.SemaphoreType.DMA,
    ],
)
def _sc_agg_l1(src_hbm, dst_hbm, ya_hbm, yb_hbm, zeros_hbm, out_hbm,
               src_v, dst_v, rows_v, acc_sh, sem):
    cid = lax.axis_index("c")
    sid = lax.axis_index("s")
    row0 = sid * ROWS_PER_SUB
    pltpu.sync_copy(zeros_hbm.at[pl.ds(row0, ROWS_PER_SUB)],
                    acc_sh.at[pl.ds(row0, ROWS_PER_SUB)])
    plsc.subcore_barrier()

    nit = jnp.where(sid == NS - 1, S_ITERS_LAST, S_ITERS)

    def edge_loop(y_hbm):
        @pl.loop(0, nit)
        def _(it):
            base = sid * E_PER_SUB + it * CHUNK
            pltpu.sync_copy(src_hbm.at[pl.ds(base, CHUNK)], src_v)
            pltpu.sync_copy(dst_hbm.at[pl.ds(base, CHUNK)], dst_v)
            pltpu.async_copy(y_hbm.at[src_v], rows_v, sem).wait()
            pltpu.sync_copy(rows_v, acc_sh.at[dst_v], add=True)

    @pl.when(cid == 0)
    def _():
        edge_loop(ya_hbm)

    @pl.when(cid == 1)
    def _():
        edge_loop(yb_hbm)

    plsc.subcore_barrier()
    pltpu.sync_copy(acc_sh.at[pl.ds(row0, ROWS_PER_SUB)],
                    out_hbm.at[cid].at[pl.ds(row0, ROWS_PER_SUB)])


@functools.partial(
    pl.kernel,
    out_type=jax.ShapeDtypeStruct((NC, NPAD, FH), jnp.float32),
    mesh=_mesh,
    compiler_params=_cparams,
    scratch_types=[
        pltpu.VMEM((CHUNK,), jnp.int32),
        pltpu.VMEM((CHUNK,), jnp.int32),
        pltpu.VMEM((CHUNK, FH), jnp.float32),
        pltpu.VMEM_SHARED((NPAD, FH), jnp.float32),
        pltpu.SemaphoreType.DMA,
    ],
)
def _sc_agg_l2(src_hbm, dst_hbm, y_hbm, zeros_hbm, out_hbm,
               src_v, dst_v, rows_v, acc_sh, sem):
    cid = lax.axis_index("c")
    sid = lax.axis_index("s")
    wid = sid * NC + cid
    row0 = sid * ROWS_PER_SUB
    pltpu.sync_copy(zeros_hbm.at[pl.ds(row0, ROWS_PER_SUB)],
                    acc_sh.at[pl.ds(row0, ROWS_PER_SUB)])
    plsc.subcore_barrier()

    nit = jnp.where(wid == NW - 1, W_ITERS_LAST, W_ITERS)

    @pl.loop(0, nit)
    def _(it):
        base = wid * E_PER_W + it * CHUNK
        pltpu.sync_copy(src_hbm.at[pl.ds(base, CHUNK)], src_v)
        pltpu.sync_copy(dst_hbm.at[pl.ds(base, CHUNK)], dst_v)
        pltpu.async_copy(y_hbm.at[src_v], rows_v, sem).wait()
        pltpu.sync_copy(rows_v, acc_sh.at[dst_v], add=True)

    plsc.subcore_barrier()
    pltpu.sync_copy(acc_sh.at[pl.ds(row0, ROWS_PER_SUB)],
                    out_hbm.at[cid].at[pl.ds(row0, ROWS_PER_SUB)])


BLK = 7168
TGRID = NPAD // BLK


def _row_spec(w):
    return pl.BlockSpec((BLK, w), lambda i: (i, 0))


def _pair_spec(w):
    return pl.BlockSpec((NC, BLK, w), lambda i: (0, i, 0))


def _rep_spec(shape):
    return pl.BlockSpec(shape, lambda i: tuple(0 for _ in shape))


def _tc_prep_body(deg_ref, x_ref, dinv_ref, xa_ref, xb_ref):
    dinv = lax.rsqrt(deg_ref[0] + deg_ref[1] + 1.0)
    dinv_ref[...] = dinv
    x = x_ref[...]
    xa_ref[...] = x[:, :FH] * dinv
    xb = jnp.concatenate(
        [x[:, FH:], jnp.zeros((BLK, 2 * FH - FEAT), jnp.float32)], axis=1)
    xb_ref[...] = xb * dinv


_tc_prep = pl.pallas_call(
    _tc_prep_body,
    grid=(TGRID,),
    in_specs=[_pair_spec(FH), _row_spec(FEAT)],
    out_specs=[_row_spec(FH), _row_spec(FH), _row_spec(FH)],
    out_shape=[jax.ShapeDtypeStruct((NPAD, FH), jnp.float32),
               jax.ShapeDtypeStruct((NPAD, FH), jnp.float32),
               jax.ShapeDtypeStruct((NPAD, FH), jnp.float32)],
)


def _tc_dense_body(acc_ref, xa_ref, xb_ref, dinv_ref, w1_ref, b1_ref,
                   w2_ref, o_ref):
    dinv = dinv_ref[...]
    agg = jnp.concatenate(
        [(acc_ref[0] + xa_ref[...]) * dinv,
         (acc_ref[1] + xb_ref[...]) * dinv], axis=1)
    h1 = jnp.maximum(
        jnp.dot(agg, w1_ref[...], preferred_element_type=jnp.float32)
        + b1_ref[...], 0.0)
    h2 = jnp.dot(h1, w2_ref[...], preferred_element_type=jnp.float32)
    o_ref[...] = h2 * dinv


_tc_dense = pl.pallas_call(
    _tc_dense_body,
    grid=(TGRID,),
    in_specs=[_pair_spec(FH), _row_spec(FH), _row_spec(FH), _row_spec(FH),
              _rep_spec((2 * FH, HID)), _rep_spec((1, HID)),
              _rep_spec((HID, FH))],
    out_specs=_row_spec(FH),
    out_shape=jax.ShapeDtypeStruct((NPAD, FH), jnp.float32),
)


def _tc_final_body(acc_ref, h2s_ref, dinv_ref, b2_ref, o_ref):
    o_ref[...] = (((acc_ref[0] + acc_ref[1] + h2s_ref[...])
                   * dinv_ref[...])[:, :OUT] + b2_ref[...])


_tc_final = pl.pallas_call(
    _tc_final_body,
    grid=(TGRID,),
    in_specs=[_pair_spec(FH), _row_spec(FH), _row_spec(FH),
              _rep_spec((1, OUT))],
    out_specs=_row_spec(OUT),
    out_shape=jax.ShapeDtypeStruct((NPAD, OUT), jnp.float32),
)


def kernel(x, edge_index, W1, b1, W2, b2):
    src = edge_index[0].astype(jnp.int32)
    dst = edge_index[1].astype(jnp.int32)

    ones_chunk = jnp.ones((CHUNK,), jnp.float32)
    z1 = jnp.zeros((NPAD,), jnp.float32)
    z16 = jnp.zeros((NPAD, FH), jnp.float32)

    deg16 = _sc_degree(dst, ones_chunk, z1)
    dinv16, xa, xb = _tc_prep(deg16, x)

    acc1 = _sc_agg_l1(src, dst, xa, xb, z16)

    w1p = jnp.pad(W1, ((0, 2 * FH - FEAT), (0, 0)))
    w2p = jnp.pad(W2, ((0, 0), (0, FH - OUT)))
    h2s = _tc_dense(acc1, xa, xb, dinv16,
                    w1p, b1.reshape(1, HID), w2p)

    acc2 = _sc_agg_l2(src, dst, h2s, z16)

    out = _tc_final(acc2, h2s, dinv16, b2.reshape(1, OUT))
    return out[:N_NODES]


# speedup vs baseline: 50.9003x; 1.0818x over previous
import functools

import jax
import jax.numpy as jnp
from jax import lax
from jax.experimental import pallas as pl
from jax.experimental.pallas import tpu as pltpu
from jax.experimental.pallas import tpu_sc as plsc

N_NODES = 100000
N_EDGES = 3200000
FEAT = 20
HID = 32
OUT = 2

NC = 2
NS = 16
NW = NC * NS
NPAD = 100352
ROWS_PER_SUB = NPAD // NS
FH = 16

CHUNK = 512
E_PER_W = 100352
W_ITERS = E_PER_W // CHUNK
W_ITERS_LAST = (N_EDGES - (NW - 1) * E_PER_W) // CHUNK
E_PER_SUB = 200704
S_ITERS = E_PER_SUB // CHUNK
S_ITERS_LAST = (N_EDGES - (NS - 1) * E_PER_SUB) // CHUNK

_mesh = plsc.VectorSubcoreMesh(core_axis_name="c", subcore_axis_name="s")
_cparams = pltpu.CompilerParams(use_tc_tiling_on_sc=False,
                                needs_layout_passes=False)


@functools.partial(
    pl.kernel,
    out_type=jax.ShapeDtypeStruct((NC, NPAD, FH), jnp.float32),
    mesh=_mesh,
    compiler_params=_cparams,
    scratch_types=[
        pltpu.VMEM((CHUNK,), jnp.int32),
        pltpu.VMEM((CHUNK,), jnp.float32),
        pltpu.VMEM((ROWS_PER_SUB,), jnp.float32),
        pltpu.VMEM((ROWS_PER_SUB, FH), jnp.float32),
        pltpu.VMEM_SHARED((NPAD,), jnp.float32),
    ],
)
def _sc_degree(dst_hbm, ones_hbm, zeros_hbm, out_hbm,
               dst_v, ones_v, slice_v, rep_v, acc_sh):
    cid = lax.axis_index("c")
    sid = lax.axis_index("s")
    wid = sid * NC + cid
    row0 = sid * ROWS_PER_SUB
    pltpu.sync_copy(zeros_hbm.at[pl.ds(row0, ROWS_PER_SUB)],
                    acc_sh.at[pl.ds(row0, ROWS_PER_SUB)])
    pltpu.sync_copy(ones_hbm, ones_v)
    plsc.subcore_barrier()

    nit = jnp.where(wid == NW - 1, W_ITERS_LAST, W_ITERS)

    @pl.loop(0, nit)
    def _(it):
        base = wid * E_PER_W + it * CHUNK
        pltpu.sync_copy(dst_hbm.at[pl.ds(base, CHUNK)], dst_v)
        pltpu.sync_copy(ones_v, acc_sh.at[dst_v], add=True)

    plsc.subcore_barrier()
    pltpu.sync_copy(acc_sh.at[pl.ds(row0, ROWS_PER_SUB)], slice_v)
    iota16 = lax.iota(jnp.int32, 16)

    @pl.loop(0, ROWS_PER_SUB, step=16)
    def _(r0):
        vals = slice_v[pl.ds(r0, 16)]
        rows = iota16 + r0
        for j in range(FH):
            plsc.store_scatter(rep_v, [rows, jnp.full((16,), j, jnp.int32)],
                               vals)

    pltpu.sync_copy(rep_v, out_hbm.at[cid].at[pl.ds(row0, ROWS_PER_SUB)])


def _edge_pipeline(src_hbm, dst_hbm, y_hbm, acc_sh, base0, nit,
                   src_a, dst_a, rows_a, sem_a, src_b, dst_b, rows_b, sem_b):
    def load_idx(i, src_v, dst_v):
        base = base0 + i * CHUNK
        pltpu.sync_copy(src_hbm.at[pl.ds(base, CHUNK)], src_v)
        pltpu.sync_copy(dst_hbm.at[pl.ds(base, CHUNK)], dst_v)

    load_idx(0, src_a, dst_a)
    pltpu.async_copy(y_hbm.at[src_a], rows_a, sem_a)

    npairs = nit // 2

    @pl.loop(0, npairs)
    def _(p):
        i1 = 2 * p + 1
        load_idx(i1, src_b, dst_b)
        pltpu.async_copy(y_hbm.at[src_b], rows_b, sem_b)
        pltpu.make_async_copy(y_hbm.at[src_a], rows_a, sem_a).wait()
        pltpu.sync_copy(rows_a, acc_sh.at[dst_a], add=True)

        @pl.when(i1 + 1 < nit)
        def _():
            load_idx(i1 + 1, src_a, dst_a)
            pltpu.async_copy(y_hbm.at[src_a], rows_a, sem_a)

        pltpu.make_async_copy(y_hbm.at[src_b], rows_b, sem_b).wait()
        pltpu.sync_copy(rows_b, acc_sh.at[dst_b], add=True)

    @pl.when(nit % 2 == 1)
    def _():
        pltpu.make_async_copy(y_hbm.at[src_a], rows_a, sem_a).wait()
        pltpu.sync_copy(rows_a, acc_sh.at[dst_a], add=True)


_AGG_SCRATCH = [
    pltpu.VMEM((CHUNK,), jnp.int32),
    pltpu.VMEM((CHUNK,), jnp.int32),
    pltpu.VMEM((CHUNK, FH), jnp.float32),
    pltpu.SemaphoreType.DMA,
    pltpu.VMEM((CHUNK,), jnp.int32),
    pltpu.VMEM((CHUNK,), jnp.int32),
    pltpu.VMEM((CHUNK, FH), jnp.float32),
    pltpu.SemaphoreType.DMA,
    pltpu.VMEM_SHARED((NPAD, FH), jnp.float32),
]


@functools.partial(
    pl.kernel,
    out_type=jax.ShapeDtypeStruct((NC, NPAD, FH), jnp.float32),
    mesh=_mesh,
    compiler_params=_cparams,
    scratch_types=_AGG_SCRATCH,
)
def _sc_agg_l1(src_hbm, dst_hbm, ya_hbm, yb_hbm, zeros_hbm, out_hbm,
               src_a, dst_a, rows_a, sem_a, src_b, dst_b, rows_b, sem_b,
               acc_sh):
    cid = lax.axis_index("c")
    sid = lax.axis_index("s")
    row0 = sid * ROWS_PER_SUB
    pltpu.sync_copy(zeros_hbm.at[pl.ds(row0, ROWS_PER_SUB)],
                    acc_sh.at[pl.ds(row0, ROWS_PER_SUB)])
    plsc.subcore_barrier()

    nit = jnp.where(sid == NS - 1, S_ITERS_LAST, S_ITERS)
    bufs = (src_a, dst_a, rows_a, sem_a, src_b, dst_b, rows_b, sem_b)

    @pl.when(cid == 0)
    def _():
        _edge_pipeline(src_hbm, dst_hbm, ya_hbm, acc_sh,
                       sid * E_PER_SUB, nit, *bufs)

    @pl.when(cid == 1)
    def _():
        _edge_pipeline(src_hbm, dst_hbm, yb_hbm, acc_sh,
                       sid * E_PER_SUB, nit, *bufs)

    plsc.subcore_barrier()
    pltpu.sync_copy(acc_sh.at[pl.ds(row0, ROWS_PER_SUB)],
                    out_hbm.at[cid].at[pl.ds(row0, ROWS_PER_SUB)])


@functools.partial(
    pl.kernel,
    out_type=jax.ShapeDtypeStruct((NC, NPAD, FH), jnp.float32),
    mesh=_mesh,
    compiler_params=_cparams,
    scratch_types=_AGG_SCRATCH,
)
def _sc_agg_l2(src_hbm, dst_hbm, y_hbm, zeros_hbm, out_hbm,
               src_a, dst_a, rows_a, sem_a, src_b, dst_b, rows_b, sem_b,
               acc_sh):
    cid = lax.axis_index("c")
    sid = lax.axis_index("s")
    wid = sid * NC + cid
    row0 = sid * ROWS_PER_SUB
    pltpu.sync_copy(zeros_hbm.at[pl.ds(row0, ROWS_PER_SUB)],
                    acc_sh.at[pl.ds(row0, ROWS_PER_SUB)])
    plsc.subcore_barrier()

    nit = jnp.where(wid == NW - 1, W_ITERS_LAST, W_ITERS)
    _edge_pipeline(src_hbm, dst_hbm, y_hbm, acc_sh, wid * E_PER_W, nit,
                   src_a, dst_a, rows_a, sem_a, src_b, dst_b, rows_b, sem_b)

    plsc.subcore_barrier()
    pltpu.sync_copy(acc_sh.at[pl.ds(row0, ROWS_PER_SUB)],
                    out_hbm.at[cid].at[pl.ds(row0, ROWS_PER_SUB)])


BLK = 7168
TGRID = NPAD // BLK


def _row_spec(w):
    return pl.BlockSpec((BLK, w), lambda i: (i, 0))


def _pair_spec(w):
    return pl.BlockSpec((NC, BLK, w), lambda i: (0, i, 0))


def _rep_spec(shape):
    return pl.BlockSpec(shape, lambda i: tuple(0 for _ in shape))


def _tc_prep_body(deg_ref, x_ref, dinv_ref, xa_ref, xb_ref):
    dinv = lax.rsqrt(deg_ref[0] + deg_ref[1] + 1.0)
    dinv_ref[...] = dinv
    x = x_ref[...]
    xa_ref[...] = x[:, :FH] * dinv
    xb = jnp.concatenate(
        [x[:, FH:], jnp.zeros((BLK, 2 * FH - FEAT), jnp.float32)], axis=1)
    xb_ref[...] = xb * dinv


_tc_prep = pl.pallas_call(
    _tc_prep_body,
    grid=(TGRID,),
    in_specs=[_pair_spec(FH), _row_spec(FEAT)],
    out_specs=[_row_spec(FH), _row_spec(FH), _row_spec(FH)],
    out_shape=[jax.ShapeDtypeStruct((NPAD, FH), jnp.float32),
               jax.ShapeDtypeStruct((NPAD, FH), jnp.float32),
               jax.ShapeDtypeStruct((NPAD, FH), jnp.float32)],
)


def _tc_dense_body(acc_ref, xa_ref, xb_ref, dinv_ref, w1_ref, b1_ref,
                   w2_ref, o_ref):
    dinv = dinv_ref[...]
    agg = jnp.concatenate(
        [(acc_ref[0] + xa_ref[...]) * dinv,
         (acc_ref[1] + xb_ref[...]) * dinv], axis=1)
    h1 = jnp.maximum(
        jnp.dot(agg, w1_ref[...], preferred_element_type=jnp.float32)
        + b1_ref[...], 0.0)
    h2 = jnp.dot(h1, w2_ref[...], preferred_element_type=jnp.float32)
    o_ref[...] = h2 * dinv


_tc_dense = pl.pallas_call(
    _tc_dense_body,
    grid=(TGRID,),
    in_specs=[_pair_spec(FH), _row_spec(FH), _row_spec(FH), _row_spec(FH),
              _rep_spec((2 * FH, HID)), _rep_spec((1, HID)),
              _rep_spec((HID, FH))],
    out_specs=_row_spec(FH),
    out_shape=jax.ShapeDtypeStruct((NPAD, FH), jnp.float32),
)


def _tc_final_body(acc_ref, h2s_ref, dinv_ref, b2_ref, o_ref):
    o_ref[...] = (((acc_ref[0] + acc_ref[1] + h2s_ref[...])
                   * dinv_ref[...])[:, :OUT] + b2_ref[...])


_tc_final = pl.pallas_call(
    _tc_final_body,
    grid=(TGRID,),
    in_specs=[_pair_spec(FH), _row_spec(FH), _row_spec(FH),
              _rep_spec((1, OUT))],
    out_specs=_row_spec(OUT),
    out_shape=jax.ShapeDtypeStruct((NPAD, OUT), jnp.float32),
)


def kernel(x, edge_index, W1, b1, W2, b2):
    src = edge_index[0].astype(jnp.int32)
    dst = edge_index[1].astype(jnp.int32)

    ones_chunk = jnp.ones((CHUNK,), jnp.float32)
    z1 = jnp.zeros((NPAD,), jnp.float32)
    z16 = jnp.zeros((NPAD, FH), jnp.float32)

    deg16 = _sc_degree(dst, ones_chunk, z1)
    dinv16, xa, xb = _tc_prep(deg16, x)

    acc1 = _sc_agg_l1(src, dst, xa, xb, z16)

    w1p = jnp.pad(W1, ((0, 2 * FH - FEAT), (0, 0)))
    w2p = jnp.pad(W2, ((0, 0), (0, FH - OUT)))
    h2s = _tc_dense(acc1, xa, xb, dinv16,
                    w1p, b1.reshape(1, HID), w2p)

    acc2 = _sc_agg_l2(src, dst, h2s, z16)

    out = _tc_final(acc2, h2s, dinv16, b2.reshape(1, OUT))
    return out[:N_NODES]


# speedup vs baseline: 70.4656x; 1.3844x over previous
import functools

import jax
import jax.numpy as jnp
from jax import lax
from jax.experimental import pallas as pl
from jax.experimental.pallas import tpu as pltpu
from jax.experimental.pallas import tpu_sc as plsc

N_NODES = 100000
N_EDGES = 3200000
FEAT = 20
HID = 32
OUT = 2

NC = 2
NS = 16
NW = NC * NS
NPAD = 100352
ROWS_PER_SUB = NPAD // NS
FH = 16
PK = NPAD * FH // 128
PK_PER_SUB = PK // NS

CHUNK = 512
E_PER_W = 100352
W_ITERS = E_PER_W // CHUNK
W_ITERS_LAST = (N_EDGES - (NW - 1) * E_PER_W) // CHUNK
E_PER_SUB = 200704
S_ITERS = E_PER_SUB // CHUNK
S_ITERS_LAST = (N_EDGES - (NS - 1) * E_PER_SUB) // CHUNK

DCHUNK = 1024
D_ITERS = E_PER_W // DCHUNK
D_ITERS_LAST = (N_EDGES - (NW - 1) * E_PER_W) // DCHUNK

_mesh = plsc.VectorSubcoreMesh(core_axis_name="c", subcore_axis_name="s")
_cparams = pltpu.CompilerParams(use_tc_tiling_on_sc=False,
                                needs_layout_passes=False)


@functools.partial(
    pl.kernel,
    out_type=jax.ShapeDtypeStruct((NC, NPAD, FH), jnp.float32),
    mesh=_mesh,
    compiler_params=_cparams,
    scratch_types=[
        pltpu.VMEM((DCHUNK,), jnp.int32),
        pltpu.VMEM((DCHUNK,), jnp.float32),
        pltpu.VMEM((ROWS_PER_SUB,), jnp.float32),
        pltpu.VMEM((ROWS_PER_SUB, FH), jnp.float32),
        pltpu.VMEM_SHARED((NPAD,), jnp.float32),
    ],
)
def _sc_degree(dst_hbm, ones_hbm, zeros_hbm, out_hbm,
               dst_v, ones_v, slice_v, rep_v, acc_sh):
    cid = lax.axis_index("c")
    sid = lax.axis_index("s")
    wid = sid * NC + cid
    row0 = sid * ROWS_PER_SUB
    pltpu.sync_copy(zeros_hbm.at[pl.ds(row0, ROWS_PER_SUB)],
                    acc_sh.at[pl.ds(row0, ROWS_PER_SUB)])
    pltpu.sync_copy(ones_hbm, ones_v)
    plsc.subcore_barrier()

    nit = jnp.where(wid == NW - 1, D_ITERS_LAST, D_ITERS)

    @pl.loop(0, nit)
    def _(it):
        base = wid * E_PER_W + it * DCHUNK
        pltpu.sync_copy(dst_hbm.at[pl.ds(base, DCHUNK)], dst_v)
        pltpu.sync_copy(ones_v, acc_sh.at[dst_v], add=True)

    plsc.subcore_barrier()
    pltpu.sync_copy(acc_sh.at[pl.ds(row0, ROWS_PER_SUB)], slice_v)
    iota16 = lax.iota(jnp.int32, 16)

    @pl.loop(0, ROWS_PER_SUB, step=16)
    def _(r0):
        vals = slice_v[pl.ds(r0, 16)]
        rows = iota16 + r0
        for j in range(FH):
            plsc.store_scatter(rep_v, [rows, jnp.full((16,), j, jnp.int32)],
                               vals)

    pltpu.sync_copy(rep_v,
                    out_hbm.at[cid].at[pl.ds(sid * ROWS_PER_SUB,
                                             ROWS_PER_SUB)])


def _edge_pipeline(src_hbm, dst_hbm, y2d, acc2d, base0, nit,
                   src_a, dst_a, rows_a, sem_a, src_b, dst_b, rows_b, sem_b):
    def load_idx(i, src_v, dst_v):
        base = base0 + i * CHUNK
        pltpu.sync_copy(src_hbm.at[pl.ds(base, CHUNK)], src_v)
        pltpu.sync_copy(dst_hbm.at[pl.ds(base, CHUNK)], dst_v)

    load_idx(0, src_a, dst_a)
    pltpu.async_copy(y2d.at[src_a], rows_a, sem_a)

    npairs = nit // 2

    @pl.loop(0, npairs)
    def _(p):
        i1 = 2 * p + 1
        load_idx(i1, src_b, dst_b)
        pltpu.async_copy(y2d.at[src_b], rows_b, sem_b)
        pltpu.make_async_copy(y2d.at[src_a], rows_a, sem_a).wait()
        pltpu.sync_copy(rows_a, acc2d.at[dst_a], add=True)

        @pl.when(i1 + 1 < nit)
        def _():
            load_idx(i1 + 1, src_a, dst_a)
            pltpu.async_copy(y2d.at[src_a], rows_a, sem_a)

        pltpu.make_async_copy(y2d.at[src_b], rows_b, sem_b).wait()
        pltpu.sync_copy(rows_b, acc2d.at[dst_b], add=True)

    @pl.when(nit % 2 == 1)
    def _():
        pltpu.make_async_copy(y2d.at[src_a], rows_a, sem_a).wait()
        pltpu.sync_copy(rows_a, acc2d.at[dst_a], add=True)


_AGG_SCRATCH = [
    pltpu.VMEM((CHUNK,), jnp.int32),
    pltpu.VMEM((CHUNK,), jnp.int32),
    pltpu.VMEM((CHUNK, FH), jnp.float32),
    pltpu.SemaphoreType.DMA,
    pltpu.VMEM((CHUNK,), jnp.int32),
    pltpu.VMEM((CHUNK,), jnp.int32),
    pltpu.VMEM((CHUNK, FH), jnp.float32),
    pltpu.SemaphoreType.DMA,
    pltpu.VMEM_SHARED((NPAD, FH), jnp.float32),
]


def _agg_epilogue(acc_sh, out_hbm, cid, sid):
    plsc.subcore_barrier()
    pltpu.sync_copy(
        acc_sh.at[pl.ds(sid * ROWS_PER_SUB, ROWS_PER_SUB)],
        out_hbm.at[cid].at[pl.ds(sid * ROWS_PER_SUB, ROWS_PER_SUB)])


def _agg_init(zeros_hbm, acc_sh, sid):
    pltpu.sync_copy(
        zeros_hbm.at[pl.ds(sid * ROWS_PER_SUB, ROWS_PER_SUB)],
        acc_sh.at[pl.ds(sid * ROWS_PER_SUB, ROWS_PER_SUB)])
    plsc.subcore_barrier()


@functools.partial(
    pl.kernel,
    out_type=jax.ShapeDtypeStruct((NC, NPAD, FH), jnp.float32),
    mesh=_mesh,
    compiler_params=_cparams,
    scratch_types=_AGG_SCRATCH,
)
def _sc_agg_l1(src_hbm, dst_hbm, ya_hbm, yb_hbm, zeros_hbm, out_hbm,
               src_a, dst_a, rows_a, sem_a, src_b, dst_b, rows_b, sem_b,
               acc_sh):
    cid = lax.axis_index("c")
    sid = lax.axis_index("s")
    _agg_init(zeros_hbm, acc_sh, sid)

    nit = jnp.where(sid == NS - 1, S_ITERS_LAST, S_ITERS)
    bufs = (src_a, dst_a, rows_a, sem_a, src_b, dst_b, rows_b, sem_b)

    @pl.when(cid == 0)
    def _():
        _edge_pipeline(src_hbm, dst_hbm, ya_hbm, acc_sh,
                       sid * E_PER_SUB, nit, *bufs)

    @pl.when(cid == 1)
    def _():
        _edge_pipeline(src_hbm, dst_hbm, yb_hbm, acc_sh,
                       sid * E_PER_SUB, nit, *bufs)

    _agg_epilogue(acc_sh, out_hbm, cid, sid)


@functools.partial(
    pl.kernel,
    out_type=jax.ShapeDtypeStruct((NC, NPAD, FH), jnp.float32),
    mesh=_mesh,
    compiler_params=_cparams,
    scratch_types=_AGG_SCRATCH,
)
def _sc_agg_l2(src_hbm, dst_hbm, y_hbm, zeros_hbm, out_hbm,
               src_a, dst_a, rows_a, sem_a, src_b, dst_b, rows_b, sem_b,
               acc_sh):
    cid = lax.axis_index("c")
    sid = lax.axis_index("s")
    wid = sid * NC + cid
    _agg_init(zeros_hbm, acc_sh, sid)

    nit = jnp.where(wid == NW - 1, W_ITERS_LAST, W_ITERS)
    _edge_pipeline(src_hbm, dst_hbm, y_hbm, acc_sh,
                   wid * E_PER_W, nit,
                   src_a, dst_a, rows_a, sem_a, src_b, dst_b, rows_b, sem_b)

    _agg_epilogue(acc_sh, out_hbm, cid, sid)


BLK = 7168
BLKP = BLK // 8
TGRID = NPAD // BLK


def _nrow_spec(w):
    return pl.BlockSpec((BLK, w), lambda i: (i, 0))


def _prow_spec():
    return pl.BlockSpec((BLKP, 128), lambda i: (i, 0))


def _pair_spec():
    return pl.BlockSpec((NC, BLKP, 128), lambda i: (0, i, 0))


def _rep_spec(shape):
    return pl.BlockSpec(shape, lambda i: tuple(0 for _ in shape))


def _tc_prep_body(deg_ref, xpa_ref, xpb_ref, dinv_ref, xa_ref, xb_ref):
    dinv = lax.rsqrt(deg_ref[0] + deg_ref[1] + 1.0)
    dinv_ref[...] = dinv
    xa_ref[...] = xpa_ref[...] * dinv
    xb_ref[...] = xpb_ref[...] * dinv


_tc_prep = pl.pallas_call(
    _tc_prep_body,
    grid=(TGRID,),
    in_specs=[_pair_spec(), _prow_spec(), _prow_spec()],
    out_specs=[_prow_spec(), _prow_spec(), _prow_spec()],
    out_shape=[jax.ShapeDtypeStruct((PK, 128), jnp.float32)] * 3,
)


def _tc_dense_body(acc_ref, xa_ref, xb_ref, dinv_ref, w1_ref, b1_ref,
                   w2_ref, o_ref):
    dinv = dinv_ref[...]
    za = (acc_ref[0] + xa_ref[...]) * dinv
    zb = (acc_ref[1] + xb_ref[...]) * dinv
    z = jnp.concatenate([za, zb], axis=1)
    h1 = jnp.maximum(
        jnp.dot(z, w1_ref[...], preferred_element_type=jnp.float32)
        + b1_ref[...], 0.0)
    h2 = jnp.dot(h1, w2_ref[...], preferred_element_type=jnp.float32)
    o_ref[...] = h2 * dinv


_tc_dense = pl.pallas_call(
    _tc_dense_body,
    grid=(TGRID,),
    in_specs=[_pair_spec(), _prow_spec(), _prow_spec(), _prow_spec(),
              _rep_spec((256, 256)), _rep_spec((1, 256)),
              _rep_spec((256, 128))],
    out_specs=_prow_spec(),
    out_shape=jax.ShapeDtypeStruct((PK, 128), jnp.float32),
)


def _tc_final_body(acc_ref, h2s_ref, dinv_ref, b2_ref, o_ref):
    o_ref[...] = ((acc_ref[0] + acc_ref[1] + h2s_ref[...])
                  * dinv_ref[...] + b2_ref[...])


_tc_final = pl.pallas_call(
    _tc_final_body,
    grid=(TGRID,),
    in_specs=[_pair_spec(), _prow_spec(), _prow_spec(),
              _rep_spec((1, 128))],
    out_specs=_prow_spec(),
    out_shape=jax.ShapeDtypeStruct((PK, 128), jnp.float32),
)


def kernel(x, edge_index, W1, b1, W2, b2):
    src = edge_index[0].astype(jnp.int32)
    dst = edge_index[1].astype(jnp.int32)

    ones_chunk = jnp.ones((DCHUNK,), jnp.float32)
    z1 = jnp.zeros((NPAD,), jnp.float32)
    zn = jnp.zeros((NPAD, FH), jnp.float32)

    xpa = jnp.pad(x[:, :FH].reshape(N_NODES * FH // 128, 128),
                  ((0, PK - N_NODES * FH // 128), (0, 0)))
    xpb = jnp.pad(
        jnp.pad(x[:, FH:], ((0, 0), (0, 2 * FH - FEAT)))
        .reshape(N_NODES * FH // 128, 128),
        ((0, PK - N_NODES * FH // 128), (0, 0)))

    deg16 = _sc_degree(dst, ones_chunk, z1)
    dinv16, xa, xb = _tc_prep(deg16.reshape(NC, PK, 128), xpa, xpb)

    acc1 = _sc_agg_l1(src, dst, xa.reshape(NPAD, FH),
                      xb.reshape(NPAD, FH), zn)

    w1p = jnp.pad(W1, ((0, 2 * FH - FEAT), (0, 0)))
    w2p = jnp.pad(W2, ((0, 0), (0, FH - OUT)))
    eye8 = jnp.eye(8, dtype=jnp.float32)
    w1big = jnp.einsum("hfgp,kK->hkfgKp", w1p.reshape(2, FH, 2, FH),
                       eye8).reshape(256, 256)
    w2big = jnp.einsum("gpo,kK->gkpKo", w2p.reshape(2, FH, FH),
                       eye8).reshape(256, 128)
    b1big = jnp.broadcast_to(b1.reshape(2, 1, FH), (2, 8, FH)).reshape(1, 256)
    b2big = jnp.broadcast_to(jnp.pad(b2, (0, FH - OUT)).reshape(1, 1, FH),
                             (1, 8, FH)).reshape(1, 128)

    h2s = _tc_dense(acc1.reshape(NC, PK, 128), xa, xb, dinv16,
                    w1big, b1big, w2big)

    acc2 = _sc_agg_l2(src, dst, h2s.reshape(NPAD, FH), zn)

    outp = _tc_final(acc2.reshape(NC, PK, 128), h2s, dinv16, b2big)
    return outp.reshape(NPAD, FH)[:N_NODES, :OUT]


# speedup vs baseline: 80.0484x; 1.1360x over previous
import functools

import jax
import jax.numpy as jnp
from jax import lax
from jax.experimental import pallas as pl
from jax.experimental.pallas import tpu as pltpu
from jax.experimental.pallas import tpu_sc as plsc

N_NODES = 100000
N_EDGES = 3200000
FEAT = 20
HID = 32
OUT = 2

NC = 2
NS = 16
NW = NC * NS
NPAD = 100352
ROWS_PER_SUB = NPAD // NS
FH = 16
PK = NPAD * FH // 128
PK_PER_SUB = PK // NS

CHUNK = 512
E_PER_W = 100352
W_ITERS = E_PER_W // CHUNK
W_ITERS_LAST = (N_EDGES - (NW - 1) * E_PER_W) // CHUNK
E_PER_SUB = 200704
S_ITERS = E_PER_SUB // CHUNK
S_ITERS_LAST = (N_EDGES - (NS - 1) * E_PER_SUB) // CHUNK

DCHUNK = 1024
D_ITERS = E_PER_W // DCHUNK
D_ITERS_LAST = (N_EDGES - (NW - 1) * E_PER_W) // DCHUNK

_mesh = plsc.VectorSubcoreMesh(core_axis_name="c", subcore_axis_name="s")
_cparams = pltpu.CompilerParams(use_tc_tiling_on_sc=False,
                                needs_layout_passes=False)


@functools.partial(
    pl.kernel,
    out_type=jax.ShapeDtypeStruct((NC, NPAD, FH), jnp.float32),
    mesh=_mesh,
    compiler_params=_cparams,
    scratch_types=[
        pltpu.VMEM((DCHUNK,), jnp.int32),
        pltpu.VMEM((DCHUNK,), jnp.float32),
        pltpu.VMEM((ROWS_PER_SUB,), jnp.float32),
        pltpu.VMEM((ROWS_PER_SUB, FH), jnp.float32),
        pltpu.VMEM_SHARED((NPAD,), jnp.float32),
    ],
)
def _sc_degree(dst_hbm, ones_hbm, zeros_hbm, out_hbm,
               dst_v, ones_v, slice_v, rep_v, acc_sh):
    cid = lax.axis_index("c")
    sid = lax.axis_index("s")
    wid = sid * NC + cid
    row0 = sid * ROWS_PER_SUB
    pltpu.sync_copy(zeros_hbm.at[pl.ds(row0, ROWS_PER_SUB)],
                    acc_sh.at[pl.ds(row0, ROWS_PER_SUB)])
    pltpu.sync_copy(ones_hbm, ones_v)
    plsc.subcore_barrier()

    nit = jnp.where(wid == NW - 1, D_ITERS_LAST, D_ITERS)

    @pl.loop(0, nit)
    def _(it):
        base = wid * E_PER_W + it * DCHUNK
        pltpu.sync_copy(dst_hbm.at[pl.ds(base, DCHUNK)], dst_v)
        pltpu.sync_copy(ones_v, acc_sh.at[dst_v], add=True)

    plsc.subcore_barrier()
    pltpu.sync_copy(acc_sh.at[pl.ds(row0, ROWS_PER_SUB)], slice_v)
    iota16 = lax.iota(jnp.int32, 16)

    @pl.loop(0, ROWS_PER_SUB, step=16)
    def _(r0):
        vals = slice_v[pl.ds(r0, 16)]
        rows = iota16 + r0
        for j in range(FH):
            plsc.store_scatter(rep_v, [rows, jnp.full((16,), j, jnp.int32)],
                               vals)

    pltpu.sync_copy(rep_v,
                    out_hbm.at[cid].at[pl.ds(sid * ROWS_PER_SUB,
                                             ROWS_PER_SUB)])


IBLK = 8


def _edge_pipeline(src2d, dst2d, y2d, acc2d, row0, nit,
                   src8, dst8, rows_a, sem_a, rows_b, sem_b):
    nbl = nit // IBLK
    rem = nit - nbl * IBLK

    @pl.loop(0, nbl)
    def _(b):
        r = row0 + b * IBLK
        pltpu.sync_copy(src2d.at[pl.ds(r, IBLK)], src8)
        pltpu.sync_copy(dst2d.at[pl.ds(r, IBLK)], dst8)
        pltpu.async_copy(y2d.at[src8.at[0]], rows_a, sem_a)

        @pl.loop(0, IBLK // 2)
        def _(p):
            k0 = 2 * p
            k1 = k0 + 1
            pltpu.async_copy(y2d.at[src8.at[k1]], rows_b, sem_b)
            pltpu.make_async_copy(y2d.at[src8.at[k0]], rows_a, sem_a).wait()
            pltpu.sync_copy(rows_a, acc2d.at[dst8.at[k0]], add=True)

            @pl.when(k1 + 1 < IBLK)
            def _():
                pltpu.async_copy(y2d.at[src8.at[k1 + 1]], rows_a, sem_a)

            pltpu.make_async_copy(y2d.at[src8.at[k1]], rows_b, sem_b).wait()
            pltpu.sync_copy(rows_b, acc2d.at[dst8.at[k1]], add=True)

    @pl.loop(0, rem)
    def _(t):
        r = row0 + nbl * IBLK + t
        pltpu.sync_copy(src2d.at[pl.ds(r, 1)], src8.at[pl.ds(0, 1)])
        pltpu.sync_copy(dst2d.at[pl.ds(r, 1)], dst8.at[pl.ds(0, 1)])
        pltpu.async_copy(y2d.at[src8.at[0]], rows_a, sem_a).wait()
        pltpu.sync_copy(rows_a, acc2d.at[dst8.at[0]], add=True)


_AGG_SCRATCH = [
    pltpu.VMEM((IBLK, CHUNK), jnp.int32),
    pltpu.VMEM((IBLK, CHUNK), jnp.int32),
    pltpu.VMEM((CHUNK, FH), jnp.float32),
    pltpu.SemaphoreType.DMA,
    pltpu.VMEM((CHUNK, FH), jnp.float32),
    pltpu.SemaphoreType.DMA,
    pltpu.VMEM_SHARED((NPAD, FH), jnp.float32),
]


def _agg_epilogue(acc_sh, out_hbm, cid, sid):
    plsc.subcore_barrier()
    pltpu.sync_copy(
        acc_sh.at[pl.ds(sid * ROWS_PER_SUB, ROWS_PER_SUB)],
        out_hbm.at[cid].at[pl.ds(sid * ROWS_PER_SUB, ROWS_PER_SUB)])


def _agg_init(zeros_hbm, acc_sh, sid):
    pltpu.sync_copy(
        zeros_hbm.at[pl.ds(sid * ROWS_PER_SUB, ROWS_PER_SUB)],
        acc_sh.at[pl.ds(sid * ROWS_PER_SUB, ROWS_PER_SUB)])
    plsc.subcore_barrier()


@functools.partial(
    pl.kernel,
    out_type=jax.ShapeDtypeStruct((NC, NPAD, FH), jnp.float32),
    mesh=_mesh,
    compiler_params=_cparams,
    scratch_types=_AGG_SCRATCH,
)
def _sc_agg_l1(src2d, dst2d, ya_hbm, yb_hbm, zeros_hbm, out_hbm,
               src8, dst8, rows_a, sem_a, rows_b, sem_b, acc_sh):
    cid = lax.axis_index("c")
    sid = lax.axis_index("s")
    _agg_init(zeros_hbm, acc_sh, sid)

    nit = jnp.where(sid == NS - 1, S_ITERS_LAST, S_ITERS)
    bufs = (src8, dst8, rows_a, sem_a, rows_b, sem_b)

    @pl.when(cid == 0)
    def _():
        _edge_pipeline(src2d, dst2d, ya_hbm, acc_sh, sid * S_ITERS, nit,
                       *bufs)

    @pl.when(cid == 1)
    def _():
        _edge_pipeline(src2d, dst2d, yb_hbm, acc_sh, sid * S_ITERS, nit,
                       *bufs)

    _agg_epilogue(acc_sh, out_hbm, cid, sid)


@functools.partial(
    pl.kernel,
    out_type=jax.ShapeDtypeStruct((NC, NPAD, FH), jnp.float32),
    mesh=_mesh,
    compiler_params=_cparams,
    scratch_types=_AGG_SCRATCH,
)
def _sc_agg_l2(src2d, dst2d, y_hbm, zeros_hbm, out_hbm,
               src8, dst8, rows_a, sem_a, rows_b, sem_b, acc_sh):
    cid = lax.axis_index("c")
    sid = lax.axis_index("s")
    wid = sid * NC + cid
    _agg_init(zeros_hbm, acc_sh, sid)

    nit = jnp.where(wid == NW - 1, W_ITERS_LAST, W_ITERS)
    _edge_pipeline(src2d, dst2d, y_hbm, acc_sh, wid * W_ITERS, nit,
                   src8, dst8, rows_a, sem_a, rows_b, sem_b)

    _agg_epilogue(acc_sh, out_hbm, cid, sid)


BLK = 7168
BLKP = BLK // 8
TGRID = NPAD // BLK


def _nrow_spec(w):
    return pl.BlockSpec((BLK, w), lambda i: (i, 0))


def _prow_spec():
    return pl.BlockSpec((BLKP, 128), lambda i: (i, 0))


def _pair_spec():
    return pl.BlockSpec((NC, BLKP, 128), lambda i: (0, i, 0))


def _rep_spec(shape):
    return pl.BlockSpec(shape, lambda i: tuple(0 for _ in shape))


def _tc_prep_body(deg_ref, xpa_ref, xpb_ref, dinv_ref, xa_ref, xb_ref):
    dinv = lax.rsqrt(deg_ref[0] + deg_ref[1] + 1.0)
    dinv_ref[...] = dinv
    xa_ref[...] = xpa_ref[...] * dinv
    xb_ref[...] = xpb_ref[...] * dinv


_tc_prep = pl.pallas_call(
    _tc_prep_body,
    grid=(TGRID,),
    in_specs=[_pair_spec(), _prow_spec(), _prow_spec()],
    out_specs=[_prow_spec(), _prow_spec(), _prow_spec()],
    out_shape=[jax.ShapeDtypeStruct((PK, 128), jnp.float32)] * 3,
)


def _tc_dense_body(acc_ref, xa_ref, xb_ref, dinv_ref, w1_ref, b1_ref,
                   w2_ref, o_ref):
    dinv = dinv_ref[...]
    za = (acc_ref[0] + xa_ref[...]) * dinv
    zb = (acc_ref[1] + xb_ref[...]) * dinv
    z = jnp.concatenate([za, zb], axis=1)
    h1 = jnp.maximum(
        jnp.dot(z, w1_ref[...], preferred_element_type=jnp.float32)
        + b1_ref[...], 0.0)
    h2 = jnp.dot(h1, w2_ref[...], preferred_element_type=jnp.float32)
    o_ref[...] = h2 * dinv


_tc_dense = pl.pallas_call(
    _tc_dense_body,
    grid=(TGRID,),
    in_specs=[_pair_spec(), _prow_spec(), _prow_spec(), _prow_spec(),
              _rep_spec((256, 256)), _rep_spec((1, 256)),
              _rep_spec((256, 128))],
    out_specs=_prow_spec(),
    out_shape=jax.ShapeDtypeStruct((PK, 128), jnp.float32),
)


def _tc_final_body(acc_ref, h2s_ref, dinv_ref, b2_ref, o_ref):
    o_ref[...] = ((acc_ref[0] + acc_ref[1] + h2s_ref[...])
                  * dinv_ref[...] + b2_ref[...])


_tc_final = pl.pallas_call(
    _tc_final_body,
    grid=(TGRID,),
    in_specs=[_pair_spec(), _prow_spec(), _prow_spec(),
              _rep_spec((1, 128))],
    out_specs=_prow_spec(),
    out_shape=jax.ShapeDtypeStruct((PK, 128), jnp.float32),
)


def kernel(x, edge_index, W1, b1, W2, b2):
    src = edge_index[0].astype(jnp.int32)
    dst = edge_index[1].astype(jnp.int32)
    src2d = src.reshape(N_EDGES // CHUNK, CHUNK)
    dst2d = dst.reshape(N_EDGES // CHUNK, CHUNK)

    ones_chunk = jnp.ones((DCHUNK,), jnp.float32)
    z1 = jnp.zeros((NPAD,), jnp.float32)
    zn = jnp.zeros((NPAD, FH), jnp.float32)

    NPK = N_NODES * FH // 128
    xflat = lax.optimization_barrier(
        x.reshape(N_NODES * FEAT // 128, 128))
    xn = xflat.reshape(N_NODES, FEAT)
    xpa = jnp.pad(xn[:, :FH].reshape(NPK, 128), ((0, PK - NPK), (0, 0)))
    xpb = jnp.pad(
        jnp.pad(xn[:, FH:], ((0, 0), (0, 2 * FH - FEAT)))
        .reshape(NPK, 128), ((0, PK - NPK), (0, 0)))

    deg16 = _sc_degree(dst, ones_chunk, z1)
    dinv16, xa, xb = _tc_prep(deg16.reshape(NC, PK, 128), xpa, xpb)

    acc1 = _sc_agg_l1(src2d, dst2d, xa.reshape(NPAD, FH),
                      xb.reshape(NPAD, FH), zn)

    w1p = jnp.pad(W1, ((0, 2 * FH - FEAT), (0, 0)))
    w2p = jnp.pad(W2, ((0, 0), (0, FH - OUT)))
    eye8 = jnp.eye(8, dtype=jnp.float32)
    w1big = jnp.einsum("hfgp,kK->hkfgKp", w1p.reshape(2, FH, 2, FH),
                       eye8).reshape(256, 256)
    w2big = jnp.einsum("gpo,kK->gkpKo", w2p.reshape(2, FH, FH),
                       eye8).reshape(256, 128)
    b1big = jnp.broadcast_to(b1.reshape(2, 1, FH), (2, 8, FH)).reshape(1, 256)
    b2big = jnp.broadcast_to(jnp.pad(b2, (0, FH - OUT)).reshape(1, 1, FH),
                             (1, 8, FH)).reshape(1, 128)

    h2s = _tc_dense(acc1.reshape(NC, PK, 128), xa, xb, dinv16,
                    w1big, b1big, w2big)

    acc2 = _sc_agg_l2(src2d, dst2d, h2s.reshape(NPAD, FH), zn)

    outp = _tc_final(acc2.reshape(NC, PK, 128), h2s, dinv16, b2big)
    return outp[:N_NODES * FH // 128].reshape(N_NODES, FH)[:, :OUT]


# speedup vs baseline: 91.1281x; 1.1384x over previous
import functools

import jax
import jax.numpy as jnp
from jax import lax
from jax.experimental import pallas as pl
from jax.experimental.pallas import tpu as pltpu
from jax.experimental.pallas import tpu_sc as plsc

N_NODES = 100000
N_EDGES = 3200000
FEAT = 20
HID = 32
OUT = 2

NC = 2
NS = 16
NW = NC * NS
NPAD = 100352
ROWS_PER_SUB = NPAD // NS
FH = 16
PK = NPAD * FH // 128
PK_PER_SUB = PK // NS

CHUNK = 512
E_PER_W = 100352
W_ITERS = E_PER_W // CHUNK
W_ITERS_LAST = (N_EDGES - (NW - 1) * E_PER_W) // CHUNK
E_PER_SUB = 200704
S_ITERS = E_PER_SUB // CHUNK
S_ITERS_LAST = (N_EDGES - (NS - 1) * E_PER_SUB) // CHUNK

DCHUNK = 1024
D_ITERS = E_PER_W // DCHUNK
D_ITERS_LAST = (N_EDGES - (NW - 1) * E_PER_W) // DCHUNK

_mesh = plsc.VectorSubcoreMesh(core_axis_name="c", subcore_axis_name="s")
_cparams = pltpu.CompilerParams(use_tc_tiling_on_sc=False,
                                needs_layout_passes=False)


@functools.partial(
    pl.kernel,
    out_type=jax.ShapeDtypeStruct((NC, NPAD, FH), jnp.float32),
    mesh=_mesh,
    compiler_params=_cparams,
    scratch_types=[
        pltpu.VMEM((DCHUNK,), jnp.int32),
        pltpu.VMEM((DCHUNK,), jnp.float32),
        pltpu.VMEM((ROWS_PER_SUB,), jnp.float32),
        pltpu.VMEM((ROWS_PER_SUB, FH), jnp.float32),
        pltpu.VMEM_SHARED((NPAD,), jnp.float32),
    ],
)
def _sc_degree(dst_hbm, ones_hbm, zeros_hbm, out_hbm,
               dst_v, ones_v, slice_v, rep_v, acc_sh):
    cid = lax.axis_index("c")
    sid = lax.axis_index("s")
    wid = sid * NC + cid
    row0 = sid * ROWS_PER_SUB
    pltpu.sync_copy(zeros_hbm.at[pl.ds(row0, ROWS_PER_SUB)],
                    acc_sh.at[pl.ds(row0, ROWS_PER_SUB)])
    pltpu.sync_copy(ones_hbm, ones_v)
    plsc.subcore_barrier()

    nit = jnp.where(wid == NW - 1, D_ITERS_LAST, D_ITERS)

    @pl.loop(0, nit)
    def _(it):
        base = wid * E_PER_W + it * DCHUNK
        pltpu.sync_copy(dst_hbm.at[pl.ds(base, DCHUNK)], dst_v)
        pltpu.sync_copy(ones_v, acc_sh.at[dst_v], add=True)

    plsc.subcore_barrier()
    pltpu.sync_copy(acc_sh.at[pl.ds(row0, ROWS_PER_SUB)], slice_v)
    iota16 = lax.iota(jnp.int32, 16)

    @pl.loop(0, ROWS_PER_SUB, step=16)
    def _(r0):
        vals = slice_v[pl.ds(r0, 16)]
        rows = iota16 + r0
        for j in range(FH):
            plsc.store_scatter(rep_v, [rows, jnp.full((16,), j, jnp.int32)],
                               vals)

    pltpu.sync_copy(rep_v,
                    out_hbm.at[cid].at[pl.ds(sid * ROWS_PER_SUB,
                                             ROWS_PER_SUB)])


IBLK = 8


def _edge_pipeline(src2d, dst2d, y2d, acc2d, row0, nit,
                   src8, dst8, rows_a, sem_a, rows_b, sem_b):
    nbl = nit // IBLK
    rem = nit - nbl * IBLK

    @pl.loop(0, nbl)
    def _(b):
        r = row0 + b * IBLK
        pltpu.sync_copy(src2d.at[pl.ds(r, IBLK)], src8)
        pltpu.sync_copy(dst2d.at[pl.ds(r, IBLK)], dst8)
        pltpu.async_copy(y2d.at[src8.at[0]], rows_a, sem_a)

        @pl.loop(0, IBLK // 2)
        def _(p):
            k0 = 2 * p
            k1 = k0 + 1
            pltpu.async_copy(y2d.at[src8.at[k1]], rows_b, sem_b)
            pltpu.make_async_copy(y2d.at[src8.at[k0]], rows_a, sem_a).wait()
            pltpu.sync_copy(rows_a, acc2d.at[dst8.at[k0]], add=True)

            @pl.when(k1 + 1 < IBLK)
            def _():
                pltpu.async_copy(y2d.at[src8.at[k1 + 1]], rows_a, sem_a)

            pltpu.make_async_copy(y2d.at[src8.at[k1]], rows_b, sem_b).wait()
            pltpu.sync_copy(rows_b, acc2d.at[dst8.at[k1]], add=True)

    @pl.loop(0, rem)
    def _(t):
        r = row0 + nbl * IBLK + t
        pltpu.sync_copy(src2d.at[pl.ds(r, 1)], src8.at[pl.ds(0, 1)])
        pltpu.sync_copy(dst2d.at[pl.ds(r, 1)], dst8.at[pl.ds(0, 1)])
        pltpu.async_copy(y2d.at[src8.at[0]], rows_a, sem_a).wait()
        pltpu.sync_copy(rows_a, acc2d.at[dst8.at[0]], add=True)


_AGG_SCRATCH = [
    pltpu.VMEM((IBLK, CHUNK), jnp.int32),
    pltpu.VMEM((IBLK, CHUNK), jnp.int32),
    pltpu.VMEM((CHUNK, FH), jnp.float32),
    pltpu.SemaphoreType.DMA,
    pltpu.VMEM((CHUNK, FH), jnp.float32),
    pltpu.SemaphoreType.DMA,
    pltpu.VMEM_SHARED((NPAD, FH), jnp.float32),
]


def _agg_epilogue(acc_sh, out_hbm, cid, sid):
    plsc.subcore_barrier()
    pltpu.sync_copy(
        acc_sh.at[pl.ds(sid * ROWS_PER_SUB, ROWS_PER_SUB)],
        out_hbm.at[cid].at[pl.ds(sid * ROWS_PER_SUB, ROWS_PER_SUB)])


def _agg_init(zeros_hbm, acc_sh, sid):
    pltpu.sync_copy(
        zeros_hbm.at[pl.ds(sid * ROWS_PER_SUB, ROWS_PER_SUB)],
        acc_sh.at[pl.ds(sid * ROWS_PER_SUB, ROWS_PER_SUB)])
    plsc.subcore_barrier()


@functools.partial(
    pl.kernel,
    out_type=jax.ShapeDtypeStruct((NC, NPAD, FH), jnp.float32),
    mesh=_mesh,
    compiler_params=_cparams,
    scratch_types=_AGG_SCRATCH,
)
def _sc_agg_l1(src2d, dst2d, ya_hbm, yb_hbm, zeros_hbm, out_hbm,
               src8, dst8, rows_a, sem_a, rows_b, sem_b, acc_sh):
    cid = lax.axis_index("c")
    sid = lax.axis_index("s")
    _agg_init(zeros_hbm, acc_sh, sid)

    nit = jnp.where(sid == NS - 1, S_ITERS_LAST, S_ITERS)
    bufs = (src8, dst8, rows_a, sem_a, rows_b, sem_b)

    @pl.when(cid == 0)
    def _():
        _edge_pipeline(src2d, dst2d, ya_hbm, acc_sh, sid * S_ITERS, nit,
                       *bufs)

    @pl.when(cid == 1)
    def _():
        _edge_pipeline(src2d, dst2d, yb_hbm, acc_sh, sid * S_ITERS, nit,
                       *bufs)

    _agg_epilogue(acc_sh, out_hbm, cid, sid)


@functools.partial(
    pl.kernel,
    out_type=jax.ShapeDtypeStruct((NC, NPAD, FH), jnp.float32),
    mesh=_mesh,
    compiler_params=_cparams,
    scratch_types=_AGG_SCRATCH,
)
def _sc_agg_l2(src2d, dst2d, y_hbm, zeros_hbm, out_hbm,
               src8, dst8, rows_a, sem_a, rows_b, sem_b, acc_sh):
    cid = lax.axis_index("c")
    sid = lax.axis_index("s")
    wid = sid * NC + cid
    _agg_init(zeros_hbm, acc_sh, sid)

    nit = jnp.where(wid == NW - 1, W_ITERS_LAST, W_ITERS)
    _edge_pipeline(src2d, dst2d, y_hbm, acc_sh, wid * W_ITERS, nit,
                   src8, dst8, rows_a, sem_a, rows_b, sem_b)

    _agg_epilogue(acc_sh, out_hbm, cid, sid)


BLK = 7168
BLKP = BLK // 8
TGRID = NPAD // BLK


def _nrow_spec(w):
    return pl.BlockSpec((BLK, w), lambda i: (i, 0))


def _prow_spec():
    return pl.BlockSpec((BLKP, 128), lambda i: (i, 0))


def _pair_spec():
    return pl.BlockSpec((NC, BLKP, 128), lambda i: (0, i, 0))


def _rep_spec(shape):
    return pl.BlockSpec(shape, lambda i: tuple(0 for _ in shape))


def _tc_prep_body(deg_ref, xpa_ref, xpb_ref, dinv_ref, xa_ref, xb_ref):
    dinv = lax.rsqrt(deg_ref[0] + deg_ref[1] + 1.0)
    dinv_ref[...] = dinv
    xa_ref[...] = xpa_ref[...] * dinv
    xb_ref[...] = xpb_ref[...] * dinv


_tc_prep = pl.pallas_call(
    _tc_prep_body,
    grid=(TGRID,),
    in_specs=[_pair_spec(), _prow_spec(), _prow_spec()],
    out_specs=[_prow_spec(), _prow_spec(), _prow_spec()],
    out_shape=[jax.ShapeDtypeStruct((PK, 128), jnp.float32)] * 3,
)


def _tc_dense_body(acc_ref, xa_ref, xb_ref, dinv_ref, w1_ref, b1_ref,
                   w2_ref, o_ref):
    dinv = dinv_ref[...]
    za = (acc_ref[0] + xa_ref[...]) * dinv
    zb = (acc_ref[1] + xb_ref[...]) * dinv
    z = jnp.concatenate([za, zb], axis=1)
    h1 = jnp.maximum(
        jnp.dot(z, w1_ref[...], preferred_element_type=jnp.float32)
        + b1_ref[...], 0.0)
    h2 = jnp.dot(h1, w2_ref[...], preferred_element_type=jnp.float32)
    o_ref[...] = h2 * dinv


_tc_dense = pl.pallas_call(
    _tc_dense_body,
    grid=(TGRID,),
    in_specs=[_pair_spec(), _prow_spec(), _prow_spec(), _prow_spec(),
              _rep_spec((256, 256)), _rep_spec((1, 256)),
              _rep_spec((256, 128))],
    out_specs=_prow_spec(),
    out_shape=jax.ShapeDtypeStruct((PK, 128), jnp.float32),
)


def _tc_final_body(acc_ref, h2s_ref, dinv_ref, b2_ref, o_ref):
    o_ref[...] = ((acc_ref[0] + acc_ref[1] + h2s_ref[...])
                  * dinv_ref[...] + b2_ref[...])


_tc_final = pl.pallas_call(
    _tc_final_body,
    grid=(TGRID,),
    in_specs=[_pair_spec(), _prow_spec(), _prow_spec(),
              _rep_spec((1, 128))],
    out_specs=_prow_spec(),
    out_shape=jax.ShapeDtypeStruct((PK, 128), jnp.float32),
)


def kernel(x, edge_index, W1, b1, W2, b2):
    src = edge_index[0].astype(jnp.int32)
    dst = edge_index[1].astype(jnp.int32)
    src2d = src.reshape(N_EDGES // CHUNK, CHUNK)
    dst2d = dst.reshape(N_EDGES // CHUNK, CHUNK)

    ones_chunk = jnp.ones((DCHUNK,), jnp.float32)
    z1 = jnp.zeros((NPAD,), jnp.float32)
    zn = jnp.zeros((NPAD, FH), jnp.float32)

    NPK = N_NODES * FH // 128
    xpa = jnp.pad(x[:, :FH].reshape(NPK, 128), ((0, PK - NPK), (0, 0)))
    xpb = jnp.pad(
        jnp.pad(x[:, FH:], ((0, 0), (0, 2 * FH - FEAT)))
        .reshape(NPK, 128), ((0, PK - NPK), (0, 0)))

    deg16 = _sc_degree(dst, ones_chunk, z1)
    dinv16, xa, xb = _tc_prep(deg16.reshape(NC, PK, 128), xpa, xpb)

    acc1 = _sc_agg_l1(src2d, dst2d, xa.reshape(NPAD, FH),
                      xb.reshape(NPAD, FH), zn)

    w1p = jnp.pad(W1, ((0, 2 * FH - FEAT), (0, 0)))
    w2p = jnp.pad(W2, ((0, 0), (0, FH - OUT)))
    eye8 = jnp.eye(8, dtype=jnp.float32)
    w1big = jnp.einsum("hfgp,kK->hkfgKp", w1p.reshape(2, FH, 2, FH),
                       eye8).reshape(256, 256)
    w2big = jnp.einsum("gpo,kK->gkpKo", w2p.reshape(2, FH, FH),
                       eye8).reshape(256, 128)
    b1big = jnp.broadcast_to(b1.reshape(2, 1, FH), (2, 8, FH)).reshape(1, 256)
    b2big = jnp.broadcast_to(jnp.pad(b2, (0, FH - OUT)).reshape(1, 1, FH),
                             (1, 8, FH)).reshape(1, 128)

    h2s = _tc_dense(acc1.reshape(NC, PK, 128), xa, xb, dinv16,
                    w1big, b1big, w2big)

    acc2 = _sc_agg_l2(src2d, dst2d, h2s.reshape(NPAD, FH), zn)

    outp = _tc_final(acc2.reshape(NC, PK, 128), h2s, dinv16, b2big)
    return (outp[:NPK].reshape(NPK, 8, FH)[:, :, :OUT]
            .reshape(N_NODES, OUT))
